# baseline (speedup 1.0000x reference)
import jax
import jax.numpy as jnp
from jax.experimental import pallas as pl
from jax.experimental.pallas import tpu as pltpu

_BATCH = 32
_VOCAB = 1_000_000
_BLOCK_V = 16384
_GRID = (_VOCAB + _BLOCK_V - 1) // _BLOCK_V

_R0 = (13, 15, 26, 6)
_R1 = (17, 29, 16, 24)


def _rotl(x, d):
    return (x << jnp.uint32(d)) | (x >> jnp.uint32(32 - d))


def _four_rounds(x0, x1, rots):
    for r in rots:
        x0 = x0 + x1
        x1 = x0 ^ _rotl(x1, r)
    return x0, x1


def _threefry_bits(i_u32):
    ks0 = jnp.uint32(0)
    ks1 = jnp.uint32(42)
    ks2 = jnp.uint32(0x1BD11BDA) ^ ks0 ^ ks1
    x0 = jnp.zeros_like(i_u32)
    x1 = i_u32 + ks1
    x0, x1 = _four_rounds(x0, x1, _R0)
    x0, x1 = x0 + ks1, x1 + ks2 + jnp.uint32(1)
    x0, x1 = _four_rounds(x0, x1, _R1)
    x0, x1 = x0 + ks2, x1 + ks0 + jnp.uint32(2)
    x0, x1 = _four_rounds(x0, x1, _R0)
    x0, x1 = x0 + ks0, x1 + ks1 + jnp.uint32(3)
    x0, x1 = _four_rounds(x0, x1, _R1)
    x0, x1 = x0 + ks1, x1 + ks2 + jnp.uint32(4)
    x0, x1 = _four_rounds(x0, x1, _R0)
    x0, x1 = x0 + ks2, x1 + ks0 + jnp.uint32(5)
    return x0 ^ x1


def _gumbel(lin_u32):
    bits = _threefry_bits(lin_u32)
    float_bits = (bits >> jnp.uint32(9)) | jnp.uint32(0x3F800000)
    f = jax.lax.bitcast_convert_type(float_bits, jnp.float32) - jnp.float32(1.0)
    minv = jnp.float32(1e-20)
    u = jnp.maximum(minv, f * (jnp.float32(1.0) - minv) + minv)
    return -jnp.log(-jnp.log(u))


def _body(logits_ref, mask_ref, out_ref, ids_ref, accv_ref, acci_ref):
    j = pl.program_id(0)

    @pl.when(j == 0)
    def _init():
        accv_ref[...] = jnp.full((_BATCH, 1), -jnp.inf, jnp.float32)
        acci_ref[...] = jnp.zeros((_BATCH, 1), jnp.int32)

    fl = logits_ref[...] * jnp.float32(2.0) + mask_ref[...]
    out_ref[...] = fl

    col = jax.lax.broadcasted_iota(jnp.int32, (_BATCH, _BLOCK_V), 1) + j * _BLOCK_V
    row = jax.lax.broadcasted_iota(jnp.int32, (_BATCH, _BLOCK_V), 0)
    lin = (row * _VOCAB + col).astype(jnp.uint32)

    cand = fl + _gumbel(lin)
    cand = jnp.where(col < _VOCAB, cand, -jnp.inf)

    m = jnp.max(cand, axis=1, keepdims=True)
    idx = jnp.min(
        jnp.where(cand == m, col, jnp.int32(2**30)), axis=1, keepdims=True
    )

    better = m > accv_ref[...]
    acci_ref[...] = jnp.where(better, idx, acci_ref[...])
    accv_ref[...] = jnp.where(better, m, accv_ref[...])

    @pl.when(j == _GRID - 1)
    def _done():
        ids_ref[...] = acci_ref[...]


def kernel(logits, prediction_mask):
    mask2d = prediction_mask.reshape(1, _VOCAB)
    final_logits, ids = pl.pallas_call(
        _body,
        grid=(_GRID,),
        in_specs=[
            pl.BlockSpec((_BATCH, _BLOCK_V), lambda j: (0, j)),
            pl.BlockSpec((1, _BLOCK_V), lambda j: (0, j)),
        ],
        out_specs=[
            pl.BlockSpec((_BATCH, _BLOCK_V), lambda j: (0, j)),
            pl.BlockSpec((_BATCH, 1), lambda j: (0, 0)),
        ],
        out_shape=[
            jax.ShapeDtypeStruct((_BATCH, _VOCAB), jnp.float32),
            jax.ShapeDtypeStruct((_BATCH, 1), jnp.int32),
        ],
        scratch_shapes=[
            pltpu.VMEM((_BATCH, 1), jnp.float32),
            pltpu.VMEM((_BATCH, 1), jnp.int32),
        ],
        compiler_params=pltpu.CompilerParams(
            dimension_semantics=("arbitrary",),
        ),
    )(logits, mask2d)
    return final_logits, ids.reshape(_BATCH)


# speedup vs baseline: 1.1617x; 1.1617x over previous
import jax
import jax.numpy as jnp
from jax import lax
from jax.experimental import pallas as pl
from jax.experimental.pallas import tpu as pltpu

_BATCH = 32
_VOCAB = 1_000_000
_BLOCK_V = 16384
_GRID = (_VOCAB + _BLOCK_V - 1) // _BLOCK_V
_CHUNK = 128
_NCH = _BLOCK_V // _CHUNK

_KS1 = 42
_KS2 = 0x1BD11BDA ^ 42
_C1 = _KS2 + 1
_C2 = 2
_C3 = _KS1 + 3
_C4 = _KS2 + 4
_C5 = 5
_R0 = (13, 15, 26, 6)
_R1 = (17, 29, 16, 24)


def _rotl(x, d):
    return (x << jnp.uint32(d)) | (x >> jnp.uint32(32 - d))


def _rounds(x0, x1, rots):
    for r in rots:
        x0 = x0 + x1
        x1 = x0 ^ _rotl(x1, r)
    return x0, x1


def _bits_from_x1(x1):
    x0 = x1
    x1 = x0 ^ _rotl(x1, 13)
    x0, x1 = _rounds(x0, x1, _R0[1:])
    x0, x1 = x0 + jnp.uint32(_KS1), x1 + jnp.uint32(_C1)
    x0, x1 = _rounds(x0, x1, _R1)
    x0, x1 = x0 + jnp.uint32(_KS2), x1 + jnp.uint32(_C2)
    x0, x1 = _rounds(x0, x1, _R0)
    x0, x1 = x0, x1 + jnp.uint32(_C3)
    x0, x1 = _rounds(x0, x1, _R1)
    x0, x1 = x0 + jnp.uint32(_KS1), x1 + jnp.uint32(_C4)
    x0, x1 = _rounds(x0, x1, _R0)
    x0, x1 = x0 + jnp.uint32(_KS2), x1 + jnp.uint32(_C5)
    return x0 ^ x1


def _gumbel_from_x1(x1):
    bits = _bits_from_x1(x1)
    float_bits = (bits >> jnp.uint32(9)) | jnp.uint32(0x3F800000)
    f = lax.bitcast_convert_type(float_bits, jnp.float32) - jnp.float32(1.0)
    u = jnp.maximum(f, jnp.float32(1e-20))
    return -jnp.log(-jnp.log(u))


def _body(logits_ref, mask_ref, out_ref, ids_ref, accv_ref, acci_ref):
    j = pl.program_id(0)

    @pl.when(j == 0)
    def _init():
        accv_ref[...] = jnp.full((_BATCH, _CHUNK), -jnp.inf, jnp.float32)
        acci_ref[...] = jnp.zeros((_BATCH, _CHUNK), jnp.int32)

    col0 = jax.lax.broadcasted_iota(jnp.int32, (_BATCH, _CHUNK), 1) + j * _BLOCK_V
    row = jax.lax.broadcasted_iota(jnp.int32, (_BATCH, _CHUNK), 0)
    x1_0 = (row * _VOCAB + col0).astype(jnp.uint32) + jnp.uint32(_KS1)

    def step(k, carry):
        x1b, col, accv, acci = carry
        off = k * _CHUNK
        fl = logits_ref[:, pl.ds(off, _CHUNK)] * jnp.float32(2.0)
        fl = fl + mask_ref[:, pl.ds(off, _CHUNK)]
        out_ref[:, pl.ds(off, _CHUNK)] = fl
        cand = fl + _gumbel_from_x1(x1b)
        cand = jnp.where(col < _VOCAB, cand, -jnp.inf)
        better = cand > accv
        acci = jnp.where(better, col, acci)
        accv = jnp.where(better, cand, accv)
        return (x1b + jnp.uint32(_CHUNK), col + _CHUNK, accv, acci)

    _, _, accv, acci = lax.fori_loop(
        0, _NCH, step, (x1_0, col0, accv_ref[...], acci_ref[...])
    )
    accv_ref[...] = accv
    acci_ref[...] = acci

    @pl.when(j == _GRID - 1)
    def _done():
        m = jnp.max(accv, axis=1, keepdims=True)
        ids_ref[...] = jnp.min(
            jnp.where(accv == m, acci, jnp.int32(2**30)), axis=1, keepdims=True
        )


def kernel(logits, prediction_mask):
    mask2d = prediction_mask.reshape(1, _VOCAB)
    final_logits, ids = pl.pallas_call(
        _body,
        grid=(_GRID,),
        in_specs=[
            pl.BlockSpec((_BATCH, _BLOCK_V), lambda j: (0, j)),
            pl.BlockSpec((1, _BLOCK_V), lambda j: (0, j)),
        ],
        out_specs=[
            pl.BlockSpec((_BATCH, _BLOCK_V), lambda j: (0, j)),
            pl.BlockSpec((_BATCH, 1), lambda j: (0, 0)),
        ],
        out_shape=[
            jax.ShapeDtypeStruct((_BATCH, _VOCAB), jnp.float32),
            jax.ShapeDtypeStruct((_BATCH, 1), jnp.int32),
        ],
        scratch_shapes=[
            pltpu.VMEM((_BATCH, _CHUNK), jnp.float32),
            pltpu.VMEM((_BATCH, _CHUNK), jnp.int32),
        ],
        compiler_params=pltpu.CompilerParams(
            dimension_semantics=("arbitrary",),
        ),
    )(logits, mask2d)
    return final_logits, ids.reshape(_BATCH)


# speedup vs baseline: 1.6554x; 1.4250x over previous
import jax
import jax.numpy as jnp
from jax import lax
from jax.experimental import pallas as pl
from jax.experimental.pallas import tpu as pltpu

_BATCH = 32
_VOCAB = 1_000_000
_BLOCK_V = 16384
_GRID = (_VOCAB + _BLOCK_V - 1) // _BLOCK_V
_PADV = _GRID * _BLOCK_V
_CHUNK = 16384
_NCH = _BLOCK_V // _CHUNK
_NSUB = _CHUNK // 128

_KS1 = 42
_KS2 = 0x1BD11BDA ^ 42
_C1 = _KS2 + 1
_C2 = 2
_C3 = _KS1 + 3
_C4 = _KS2 + 4
_C5 = 5
_R0 = (13, 15, 26, 6)
_R1 = (17, 29, 16, 24)


def _rotl(x, d):
    return (x << jnp.uint32(d)) | (x >> jnp.uint32(32 - d))


def _rounds(x0, x1, rots):
    for r in rots:
        x0 = x0 + x1
        x1 = x0 ^ _rotl(x1, r)
    return x0, x1


def _bits_from_x1(x1):
    x0 = x1
    x1 = x0 ^ _rotl(x1, 13)
    x0, x1 = _rounds(x0, x1, _R0[1:])
    x0, x1 = x0 + jnp.uint32(_KS1), x1 + jnp.uint32(_C1)
    x0, x1 = _rounds(x0, x1, _R1)
    x0, x1 = x0 + jnp.uint32(_KS2), x1 + jnp.uint32(_C2)
    x0, x1 = _rounds(x0, x1, _R0)
    x0, x1 = x0, x1 + jnp.uint32(_C3)
    x0, x1 = _rounds(x0, x1, _R1)
    x0, x1 = x0 + jnp.uint32(_KS1), x1 + jnp.uint32(_C4)
    x0, x1 = _rounds(x0, x1, _R0)
    x0, x1 = x0 + jnp.uint32(_KS2), x1 + jnp.uint32(_C5)
    return x0 ^ x1


def _gumbel_from_x1(x1):
    bits = _bits_from_x1(x1)
    float_bits = (bits >> jnp.uint32(9)) | jnp.uint32(0x3F800000)
    f = lax.bitcast_convert_type(float_bits, jnp.float32) - jnp.float32(1.0)
    u = jnp.maximum(f, jnp.float32(1e-20))
    return -jnp.log(-jnp.log(u))


def _body(logits_ref, mask_ref, out_ref, ids_ref, accv_ref, acci_ref):
    j = pl.program_id(0)

    @pl.when(j == 0)
    def _init():
        accv_ref[...] = jnp.full((_BATCH, 128), -jnp.inf, jnp.float32)
        acci_ref[...] = jnp.zeros((_BATCH, 128), jnp.int32)

    colb = jax.lax.broadcasted_iota(jnp.int32, (_BATCH, _CHUNK), 1)
    rowb = jax.lax.broadcasted_iota(jnp.int32, (_BATCH, _CHUNK), 0)
    linbase = (rowb * _VOCAB + colb + j * _BLOCK_V).astype(jnp.uint32)

    def step(k, carry):
        accv, acci = carry
        off = k * _CHUNK
        lin = linbase + off.astype(jnp.uint32)
        fl = logits_ref[:, pl.ds(off, _CHUNK)] * jnp.float32(2.0)
        fl = fl + mask_ref[:, pl.ds(off, _CHUNK)]
        out_ref[:, pl.ds(off, _CHUNK)] = fl
        cand = fl + _gumbel_from_x1(lin + jnp.uint32(_KS1))
        lin_i = lin.astype(jnp.int32)
        for s in range(_NSUB):
            c = cand[:, s * 128:(s + 1) * 128]
            li = lin_i[:, s * 128:(s + 1) * 128]
            better = c > accv
            acci = jnp.where(better, li, acci)
            accv = jnp.where(better, c, accv)
        return (accv, acci)

    accv, acci = lax.fori_loop(
        0, _NCH, step, (accv_ref[...], acci_ref[...])
    )
    accv_ref[...] = accv
    acci_ref[...] = acci

    @pl.when(j == _GRID - 1)
    def _done():
        row = jax.lax.broadcasted_iota(jnp.int32, (_BATCH, 128), 0)
        col = acci - row * _VOCAB
        m = jnp.max(accv, axis=1, keepdims=True)
        ids_ref[...] = jnp.min(
            jnp.where(accv == m, col, jnp.int32(2**30)), axis=1, keepdims=True
        )


def kernel(logits, prediction_mask):
    mask2d = jnp.concatenate(
        [
            prediction_mask.reshape(1, _VOCAB),
            jnp.full((1, _PADV - _VOCAB), -jnp.inf, jnp.float32),
        ],
        axis=1,
    )
    final_logits, ids = pl.pallas_call(
        _body,
        grid=(_GRID,),
        in_specs=[
            pl.BlockSpec((_BATCH, _BLOCK_V), lambda j: (0, j)),
            pl.BlockSpec((1, _BLOCK_V), lambda j: (0, j)),
        ],
        out_specs=[
            pl.BlockSpec((_BATCH, _BLOCK_V), lambda j: (0, j)),
            pl.BlockSpec((_BATCH, 1), lambda j: (0, 0)),
        ],
        out_shape=[
            jax.ShapeDtypeStruct((_BATCH, _VOCAB), jnp.float32),
            jax.ShapeDtypeStruct((_BATCH, 1), jnp.int32),
        ],
        scratch_shapes=[
            pltpu.VMEM((_BATCH, 128), jnp.float32),
            pltpu.VMEM((_BATCH, 128), jnp.int32),
        ],
        compiler_params=pltpu.CompilerParams(
            dimension_semantics=("arbitrary",),
        ),
    )(logits, mask2d)
    return final_logits, ids.reshape(_BATCH)
